# baseline (speedup 1.0000x reference)
import functools

import jax
import jax.numpy as jnp
import numpy as np
from jax.experimental import pallas as pl
from jax.experimental.pallas import tpu as pltpu

_MASK_RATIO = float(np.random.default_rng(0).uniform(0.1, 0.5))

_BLK = 2048


def _mask_mul_kernel(sites_ref, x_ref, o_ref, *, n_sites, w):
    sites = sites_ref[0, :].reshape(n_sites, 1)
    cols = jax.lax.broadcasted_iota(jnp.int32, (1, w), 1)
    hit = jnp.any(sites == cols, axis=0, keepdims=True)
    o_ref[...] = jnp.where(hit, jnp.float32(0), x_ref[...])


def kernel(x):
    b, c, h, w = x.shape
    n_sites = int(_MASK_RATIO * w)
    perm = jax.random.permutation(jax.random.key(1), w)
    sites = perm[:n_sites].astype(jnp.int32).reshape(1, n_sites)

    rows = b * c * h
    x2 = x.reshape(rows, w)
    grid = rows // _BLK

    out = pl.pallas_call(
        functools.partial(_mask_mul_kernel, n_sites=n_sites, w=w),
        grid=(grid,),
        in_specs=[
            pl.BlockSpec((1, n_sites), lambda i: (0, 0)),
            pl.BlockSpec((_BLK, w), lambda i: (i, 0)),
        ],
        out_specs=pl.BlockSpec((_BLK, w), lambda i: (i, 0)),
        out_shape=jax.ShapeDtypeStruct((rows, w), x.dtype),
        compiler_params=pltpu.CompilerParams(
            dimension_semantics=("arbitrary",),
        ),
    )(sites, x2)
    return out.reshape(b, c, h, w)


# speedup vs baseline: 1.0042x; 1.0042x over previous
import functools

import jax
import jax.numpy as jnp
import numpy as np
from jax.experimental import pallas as pl
from jax.experimental.pallas import tpu as pltpu

_MASK_RATIO = float(np.random.default_rng(0).uniform(0.1, 0.5))

_BLK = 2048


def _mask_mul_kernel(sites_ref, x_ref, o_ref, *, n_sites, w):
    sites = sites_ref[0, :].reshape(n_sites, 1)
    cols = jax.lax.broadcasted_iota(jnp.int32, (1, w), 1)
    hit = jnp.any(sites == cols, axis=0, keepdims=True)
    o_ref[...] = jnp.where(hit[None, None], jnp.float32(0), x_ref[...])


_CB = 8


def kernel(x):
    b, c, h, w = x.shape
    n_sites = int(_MASK_RATIO * w)
    perm = jax.random.permutation(jax.random.key(1), w)
    sites = perm[:n_sites].astype(jnp.int32).reshape(1, n_sites)

    return pl.pallas_call(
        functools.partial(_mask_mul_kernel, n_sites=n_sites, w=w),
        grid=(b, c // _CB),
        in_specs=[
            pl.BlockSpec((1, n_sites), lambda i, j: (0, 0)),
            pl.BlockSpec((1, _CB, h, w), lambda i, j: (i, j, 0, 0)),
        ],
        out_specs=pl.BlockSpec((1, _CB, h, w), lambda i, j: (i, j, 0, 0)),
        out_shape=jax.ShapeDtypeStruct((b, c, h, w), x.dtype),
        compiler_params=pltpu.CompilerParams(
            dimension_semantics=("arbitrary", "arbitrary"),
        ),
    )(sites, x)


# speedup vs baseline: 1.8667x; 1.8589x over previous
import functools

import jax
import jax.numpy as jnp
import numpy as np
from jax.experimental import pallas as pl
from jax.experimental.pallas import tpu as pltpu

_MASK_RATIO = float(np.random.default_rng(0).uniform(0.1, 0.5))

_BLK = 2048


def _mask_mul_kernel(sites_ref, x_ref, o_ref, *, n_sites, h, w):
    sites = sites_ref[0, :].reshape(n_sites, 1)
    cols = jax.lax.broadcasted_iota(jnp.int32, (1, w), 1)
    hit = jnp.any(sites == cols, axis=0, keepdims=True)
    maskrow = jnp.where(hit, jnp.float32(0), jnp.float32(1))
    mask2d = jnp.broadcast_to(maskrow, (h, w))
    o_ref[...] = x_ref[...] * mask2d[None, None]


_CB = 8


def kernel(x):
    b, c, h, w = x.shape
    n_sites = int(_MASK_RATIO * w)
    perm = jax.random.permutation(jax.random.key(1), w)
    sites = perm[:n_sites].astype(jnp.int32).reshape(1, n_sites)

    return pl.pallas_call(
        functools.partial(_mask_mul_kernel, n_sites=n_sites, h=h, w=w),
        grid=(b, c // _CB),
        in_specs=[
            pl.BlockSpec((1, n_sites), lambda i, j: (0, 0)),
            pl.BlockSpec((1, _CB, h, w), lambda i, j: (i, j, 0, 0)),
        ],
        out_specs=pl.BlockSpec((1, _CB, h, w), lambda i, j: (i, j, 0, 0)),
        out_shape=jax.ShapeDtypeStruct((b, c, h, w), x.dtype),
        compiler_params=pltpu.CompilerParams(
            dimension_semantics=("arbitrary", "arbitrary"),
        ),
    )(sites, x)


# speedup vs baseline: 1.8802x; 1.0072x over previous
import functools

import jax
import jax.numpy as jnp
import numpy as np
from jax.experimental import pallas as pl
from jax.experimental.pallas import tpu as pltpu

_MASK_RATIO = float(np.random.default_rng(0).uniform(0.1, 0.5))

_BLK = 2048


def _mask_mul_kernel(sites_ref, x_ref, o_ref, *, n_sites, h, w):
    sites = sites_ref[0, :].reshape(n_sites, 1)
    cols = jax.lax.broadcasted_iota(jnp.int32, (1, w), 1)
    hit = jnp.any(sites == cols, axis=0, keepdims=True)
    maskrow = jnp.where(hit, jnp.float32(0), jnp.float32(1))
    mask2d = jnp.broadcast_to(maskrow, (h, w))
    o_ref[...] = x_ref[...] * mask2d[None, None]


_CB = 16


def kernel(x):
    b, c, h, w = x.shape
    n_sites = int(_MASK_RATIO * w)
    perm = jax.random.permutation(jax.random.key(1), w)
    sites = perm[:n_sites].astype(jnp.int32).reshape(1, n_sites)

    return pl.pallas_call(
        functools.partial(_mask_mul_kernel, n_sites=n_sites, h=h, w=w),
        grid=(b, c // _CB),
        in_specs=[
            pl.BlockSpec((1, n_sites), lambda i, j: (0, 0)),
            pl.BlockSpec((1, _CB, h, w), lambda i, j: (i, j, 0, 0)),
        ],
        out_specs=pl.BlockSpec((1, _CB, h, w), lambda i, j: (i, j, 0, 0)),
        out_shape=jax.ShapeDtypeStruct((b, c, h, w), x.dtype),
        compiler_params=pltpu.CompilerParams(
            dimension_semantics=("arbitrary", "arbitrary"),
        ),
    )(sites, x)


# speedup vs baseline: 1.8898x; 1.0051x over previous
import functools

import jax
import jax.numpy as jnp
import numpy as np
from jax.experimental import pallas as pl
from jax.experimental.pallas import tpu as pltpu

_MASK_RATIO = float(np.random.default_rng(0).uniform(0.1, 0.5))

_BLK = 2048


def _mask_mul_kernel(sites_ref, x_ref, o_ref, *, n_sites, h, w):
    sites = sites_ref[0, :].reshape(n_sites, 1)
    cols = jax.lax.broadcasted_iota(jnp.int32, (1, w), 1)
    hit = jnp.any(sites == cols, axis=0, keepdims=True)
    maskrow = jnp.where(hit, jnp.float32(0), jnp.float32(1))
    mask2d = jnp.broadcast_to(maskrow, (h, w))
    o_ref[...] = x_ref[...] * mask2d[None, None]


_CB = 24


def kernel(x):
    b, c, h, w = x.shape
    n_sites = int(_MASK_RATIO * w)
    perm = jax.random.permutation(jax.random.key(1), w)
    sites = perm[:n_sites].astype(jnp.int32).reshape(1, n_sites)

    return pl.pallas_call(
        functools.partial(_mask_mul_kernel, n_sites=n_sites, h=h, w=w),
        grid=(b, c // _CB),
        in_specs=[
            pl.BlockSpec((1, n_sites), lambda i, j: (0, 0)),
            pl.BlockSpec((1, _CB, h, w), lambda i, j: (i, j, 0, 0)),
        ],
        out_specs=pl.BlockSpec((1, _CB, h, w), lambda i, j: (i, j, 0, 0)),
        out_shape=jax.ShapeDtypeStruct((b, c, h, w), x.dtype),
        compiler_params=pltpu.CompilerParams(
            dimension_semantics=("arbitrary", "arbitrary"),
        ),
    )(sites, x)
